# baseline (speedup 1.0000x reference)
import jax
import jax.numpy as jnp
from jax.experimental import pallas as pl
from jax.experimental.pallas import tpu as pltpu

TAU_ = 1.0
C_ = 100
CPAD_ = 128
B_ = 2048
D_ = 1024
BLK_ = 256
NB_ = B_ // BLK_

_DN_ROWS = (((0,), (0,)), ((), ()))
_DN_FEAT = (((1,), (1,)), ((), ()))


def _loss_kernel(z1_ref, z2_ref, lab_ref, out_ref,
                 zb1_s, zb2_s, inv1_s, inv2_s,
                 sums_s, cnt_s, sumsb_s, invc_s, loss_s):
    f32 = jnp.float32
    bf16 = jnp.bfloat16
    t = pl.program_id(0)
    j = pl.program_id(1)

    lab = lab_ref[...]
    col = jax.lax.broadcasted_iota(jnp.int32, (BLK_, CPAD_), 1)
    oh = (lab == col).astype(f32)

    @pl.when(jnp.logical_and(t == 0, j == 0))
    def _init():
        sums_s[...] = jnp.zeros_like(sums_s)
        cnt_s[...] = jnp.zeros_like(cnt_s)
        loss_s[...] = jnp.zeros_like(loss_s)

    @pl.when(t == 0)
    def _phase0():
        def prep(z_ref, zb_s, inv_s):
            z = z_ref[...]
            ss = jnp.sum(z * z, axis=1, keepdims=True)
            inv = jax.lax.rsqrt(jnp.maximum(ss, 1e-24))
            zb = z.astype(bf16)
            zb_s[pl.ds(j * BLK_, BLK_), :] = zb
            inv_s[pl.ds(j * BLK_, BLK_), :] = inv
            return zb, inv

        zb1, inv1 = prep(z1_ref, zb1_s, inv1_s)
        zb2, inv2 = prep(z2_ref, zb2_s, inv2_s)
        ohs1 = (oh * inv1).astype(bf16)
        ohs2 = (oh * inv2).astype(bf16)
        part = (jax.lax.dot_general(ohs1, zb1, _DN_ROWS,
                                    preferred_element_type=f32)
                + jax.lax.dot_general(ohs2, zb2, _DN_ROWS,
                                      preferred_element_type=f32))
        sums_s[...] += part
        cnt_s[...] += jnp.sum(oh, axis=0, keepdims=True)

    @pl.when(t == 1)
    def _phase1():
        @pl.when(j == 0)
        def _finalize():
            sumsb_s[...] = sums_s[...].astype(bf16)
            invc_s[...] = 1.0 / jnp.maximum(2.0 * cnt_s[...], 1.0)

        sumsb = sumsb_s[...]
        invc = invc_s[...] * (1.0 / TAU_)
        vmask = (jax.lax.broadcasted_iota(jnp.int32, (1, CPAD_), 1)
                 < C_).astype(f32)

        def view_loss(zb_s, inv_s):
            zb = zb_s[pl.ds(j * BLK_, BLK_), :]
            inv = inv_s[pl.ds(j * BLK_, BLK_), :]
            simr = jax.lax.dot_general(zb, sumsb, _DN_FEAT,
                                       preferred_element_type=f32)
            sim = simr * invc * inv
            p = jnp.sum(sim * oh, axis=1, keepdims=True)
            s = jnp.sum(jnp.exp(sim - p) * vmask, axis=1, keepdims=True)
            return jnp.log(s) - p

        part = jnp.sum(view_loss(zb1_s, inv1_s) + view_loss(zb2_s, inv2_s),
                       axis=0, keepdims=True)
        loss_s[...] += part

    @pl.when(jnp.logical_and(t == 1, j == NB_ - 1))
    def _out():
        out_ref[...] = loss_s[...] * (1.0 / (2.0 * B_))


def kernel(z1_features, z2_features, labels):
    lab2d = labels.astype(jnp.int32).reshape(B_, 1)
    out = pl.pallas_call(
        _loss_kernel,
        grid=(2, NB_),
        in_specs=[
            pl.BlockSpec((BLK_, D_),
                         lambda t, j: (jnp.where(t == 0, j, NB_ - 1), 0)),
            pl.BlockSpec((BLK_, D_),
                         lambda t, j: (jnp.where(t == 0, j, NB_ - 1), 0)),
            pl.BlockSpec((BLK_, 1), lambda t, j: (j, 0)),
        ],
        out_specs=pl.BlockSpec((1, 1), lambda t, j: (0, 0)),
        out_shape=jax.ShapeDtypeStruct((1, 1), jnp.float32),
        scratch_shapes=[
            pltpu.VMEM((B_, D_), jnp.bfloat16),
            pltpu.VMEM((B_, D_), jnp.bfloat16),
            pltpu.VMEM((B_, 1), jnp.float32),
            pltpu.VMEM((B_, 1), jnp.float32),
            pltpu.VMEM((CPAD_, D_), jnp.float32),
            pltpu.VMEM((1, CPAD_), jnp.float32),
            pltpu.VMEM((CPAD_, D_), jnp.bfloat16),
            pltpu.VMEM((1, CPAD_), jnp.float32),
            pltpu.VMEM((1, 1), jnp.float32),
        ],
        compiler_params=pltpu.CompilerParams(
            dimension_semantics=("arbitrary", "arbitrary"),
            vmem_limit_bytes=100 * 1024 * 1024,
        ),
    )(z1_features, z2_features, lab2d)
    return out[0, 0]


# speedup vs baseline: 1.2879x; 1.2879x over previous
import jax
import jax.numpy as jnp
from jax.experimental import pallas as pl
from jax.experimental.pallas import tpu as pltpu

TAU_ = 1.0
C_ = 100
CPAD_ = 128
B_ = 2048
D_ = 1024
BLK_ = 512
NB_ = B_ // BLK_

_DN_ROWS = (((0,), (0,)), ((), ()))
_DN_FEAT = (((1,), (1,)), ((), ()))


def _loss_kernel(z1_ref, z2_ref, lab_ref, out_ref,
                 nb1_s, nb2_s, sums_s, cnt_s, sumsb_s, invc_s, loss_s):
    f32 = jnp.float32
    bf16 = jnp.bfloat16
    t = pl.program_id(0)
    j = pl.program_id(1)

    lab = lab_ref[...]
    col = jax.lax.broadcasted_iota(jnp.int32, (BLK_, CPAD_), 1)
    oh = (lab == col).astype(f32)

    @pl.when(jnp.logical_and(t == 0, j == 0))
    def _init():
        sums_s[...] = jnp.zeros_like(sums_s)
        cnt_s[...] = jnp.zeros_like(cnt_s)
        loss_s[...] = jnp.zeros_like(loss_s)

    @pl.when(t == 0)
    def _phase0():
        ohb = oh.astype(bf16)

        def prep(z_ref, nb_s):
            z = z_ref[...]
            ss = jnp.sum(z * z, axis=1, keepdims=True)
            inv = jax.lax.rsqrt(jnp.maximum(ss, 1e-24))
            nb = (z * inv).astype(bf16)
            nb_s[pl.ds(j * BLK_, BLK_), :] = nb
            return nb

        nb1 = prep(z1_ref, nb1_s)
        nb2 = prep(z2_ref, nb2_s)
        part = (jax.lax.dot_general(ohb, nb1, _DN_ROWS,
                                    preferred_element_type=f32)
                + jax.lax.dot_general(ohb, nb2, _DN_ROWS,
                                      preferred_element_type=f32))
        sums_s[...] += part
        cnt_s[...] += jnp.sum(oh, axis=0, keepdims=True)

    @pl.when(t == 1)
    def _phase1():
        @pl.when(j == 0)
        def _finalize():
            sumsb_s[...] = sums_s[...].astype(bf16)
            invc_s[...] = 1.0 / jnp.maximum(2.0 * cnt_s[...], 1.0)

        sumsb = sumsb_s[...]
        invc = invc_s[...] * (1.0 / TAU_)
        vmask = (jax.lax.broadcasted_iota(jnp.int32, (1, CPAD_), 1)
                 < C_).astype(f32)

        def view_loss(nb_s):
            nb = nb_s[pl.ds(j * BLK_, BLK_), :]
            simr = jax.lax.dot_general(nb, sumsb, _DN_FEAT,
                                       preferred_element_type=f32)
            sim = simr * invc
            p = jnp.sum(sim * oh, axis=1, keepdims=True)
            s = jnp.sum(jnp.exp(sim - p) * vmask, axis=1, keepdims=True)
            return jnp.log(s) - p

        part = jnp.sum(view_loss(nb1_s) + view_loss(nb2_s),
                       axis=0, keepdims=True)
        loss_s[...] += part

    @pl.when(jnp.logical_and(t == 1, j == NB_ - 1))
    def _out():
        out_ref[...] = loss_s[...] * (1.0 / (2.0 * B_))


def kernel(z1_features, z2_features, labels):
    lab2d = labels.astype(jnp.int32).reshape(B_, 1)
    out = pl.pallas_call(
        _loss_kernel,
        grid=(2, NB_),
        in_specs=[
            pl.BlockSpec((BLK_, D_),
                         lambda t, j: (jnp.where(t == 0, j, NB_ - 1), 0)),
            pl.BlockSpec((BLK_, D_),
                         lambda t, j: (jnp.where(t == 0, j, NB_ - 1), 0)),
            pl.BlockSpec((BLK_, 1), lambda t, j: (j, 0)),
        ],
        out_specs=pl.BlockSpec((1, 1), lambda t, j: (0, 0)),
        out_shape=jax.ShapeDtypeStruct((1, 1), jnp.float32),
        scratch_shapes=[
            pltpu.VMEM((B_, D_), jnp.bfloat16),
            pltpu.VMEM((B_, D_), jnp.bfloat16),
            pltpu.VMEM((CPAD_, D_), jnp.float32),
            pltpu.VMEM((1, CPAD_), jnp.float32),
            pltpu.VMEM((CPAD_, D_), jnp.bfloat16),
            pltpu.VMEM((1, CPAD_), jnp.float32),
            pltpu.VMEM((1, 1), jnp.float32),
        ],
        compiler_params=pltpu.CompilerParams(
            dimension_semantics=("arbitrary", "arbitrary"),
            vmem_limit_bytes=100 * 1024 * 1024,
        ),
    )(z1_features, z2_features, lab2d)
    return out[0, 0]
